# baseline (speedup 1.0000x reference)
import functools

import jax
import jax.numpy as jnp
from jax import lax
from jax.experimental import pallas as pl
from jax.experimental.pallas import tpu as pltpu
from jax.experimental.pallas import tpu_sc as plsc

NC, NS = 2, 16
NW = NC * NS
B = 16384
C = 128
S = B // NW
NCH = S // C
ED = 64
PD = 32


def _sc_gather_all(idx2, tables):
    mesh = plsc.VectorSubcoreMesh(
        core_axis_name="c", subcore_axis_name="s",
        num_cores=NC, num_subcores=NS)
    dims = (ED, ED, ED, PD, PD, PD, PD, PD, PD)
    tsel = (0, 1, 1, 2, 2, 3, 3, 4, 4)
    dts = tuple(tables[t].dtype for t in tsel)

    out_type = [jax.ShapeDtypeStruct((B, d), dt) for d, dt in zip(dims, dts)]
    scratch = (
        [pltpu.VMEM((NCH, C), jnp.int32) for _ in range(9)]
        + [pltpu.VMEM((C, d), dt) for d, dt in zip(dims, dts)]
        + [pltpu.SemaphoreType.DMA]
    )

    @functools.partial(
        pl.kernel, mesh=mesh, out_type=out_type, scratch_types=scratch,
        name="bpr_sc_gather",
        compiler_params=pltpu.CompilerParams(use_tc_tiling_on_sc=False),
    )
    def k(*refs):
        idx_hbm = refs[0:9]
        tab_hbm = refs[9:14]
        out_hbm = refs[14:23]
        idx_v = refs[23:32]
        row_v = refs[32:41]
        sem = refs[41]

        wid = lax.axis_index("s") * NC + lax.axis_index("c")
        row0 = wid * NCH

        for t in range(9):
            pltpu.sync_copy(idx_hbm[t].at[pl.ds(row0, NCH)], idx_v[t])

        def chunk(g):
            off = pl.multiple_of((row0 + g) * C, C)
            handles = [
                pltpu.async_copy(
                    tab_hbm[tsel[t]].at[idx_v[t].at[g]], row_v[t], sem)
                for t in range(9)
            ]
            for h in handles:
                h.wait()
            for t in range(9):
                pltpu.sync_copy(row_v[t], out_hbm[t].at[pl.ds(off, C)])

        pl.loop(0, NCH)(chunk)

    return k(*idx2, *tables)


_BN = 4096


def _conv_body(u_ref, i_ref, ou_ref, oi_ref):
    eye = jnp.eye(ED, dtype=jnp.float32)
    tr = lambda x: lax.dot_general(x, eye, (((0,), (0,)), ((), ())))
    ou_ref[...] = tr(u_ref[...])
    oi_ref[...] = tr(i_ref[...])


def _tc_convert(user_table, item_table):
    n = user_table.shape[0]
    grid = (n + _BN - 1) // _BN
    specs = [pl.BlockSpec((ED, _BN), lambda i: (0, i))] * 2
    out = pl.pallas_call(
        _conv_body,
        grid=(grid,),
        in_specs=specs,
        out_specs=[pl.BlockSpec((_BN, ED), lambda i: (i, 0))] * 2,
        out_shape=[jax.ShapeDtypeStruct((n, ED), jnp.float32)] * 2,
        compiler_params=pltpu.CompilerParams(
            fuse_transposed_lhs_in_matmul=True),
    )(user_table.T, item_table.T)
    return out


_BM = 2048


def _tc_body(u_ref, pi_ref, ni_ref, pc_ref, nc_ref, ppt_ref, npt_ref,
             pv_ref, nv_ref, w_ref, b_ref, o_ref):
    dot = functools.partial(lax.dot, preferred_element_type=jnp.float32)
    w1 = w_ref[0:ED, :]
    w2 = w_ref[ED:ED + PD, :]
    w3 = w_ref[ED + PD:ED + 2 * PD, :]
    w4 = w_ref[ED + 2 * PD:ED + 3 * PD, :]
    bb = b_ref[0:1, :]
    pre_p = (dot(pi_ref[...], w1) + dot(pc_ref[...], w2)
             + dot(ppt_ref[...], w3) + dot(pv_ref[...], w4) + bb)
    pre_n = (dot(ni_ref[...], w1) + dot(nc_ref[...], w2)
             + dot(npt_ref[...], w3) + dot(nv_ref[...], w4) + bb)
    u = u_ref[...].astype(jnp.float32)
    s = (jnp.sum(u * jnp.maximum(pre_p, 0.0), axis=1)
         - jnp.sum(u * jnp.maximum(pre_n, 0.0), axis=1))
    o_ref[...] = s[None, None, :]


def _tc_score(rows, W, b):
    grid = B // _BM
    in_specs = [
        pl.BlockSpec((_BM, r.shape[1]), lambda i: (i, 0)) for r in rows
    ] + [
        pl.BlockSpec((ED + 3 * PD, ED), lambda i: (0, 0)),
        pl.BlockSpec((1, ED), lambda i: (0, 0)),
    ]
    out = pl.pallas_call(
        _tc_body,
        grid=(grid,),
        in_specs=in_specs,
        out_specs=pl.BlockSpec((1, 1, _BM), lambda i: (i, 0, 0)),
        out_shape=jax.ShapeDtypeStruct((grid, 1, _BM), jnp.float32),
    )(*rows, W, b.reshape(1, ED))
    return out.reshape(B)


def kernel(user_ids, pos_item_ids, neg_item_ids, pos_cat, neg_cat,
           pos_prop_type, pos_prop_value, neg_prop_type, neg_prop_value,
           user_table, item_table, cat_table, ptype_table, pval_table, W, b):
    idx = [user_ids, pos_item_ids, neg_item_ids, pos_cat, neg_cat,
           pos_prop_type, neg_prop_type, pos_prop_value, neg_prop_value]
    idx2 = [i.astype(jnp.int32).reshape(B // C, C) for i in idx]
    user_bf, item_bf = _tc_convert(user_table, item_table)
    tables = [user_bf, item_bf, cat_table, ptype_table, pval_table]
    rows = _sc_gather_all(idx2, tables)
    return _tc_score(rows, W, b)


# speedup vs baseline: 2.1320x; 2.1320x over previous
import functools

import jax
import jax.numpy as jnp
from jax import lax
from jax.experimental import pallas as pl
from jax.experimental.pallas import tpu as pltpu
from jax.experimental.pallas import tpu_sc as plsc

NC, NS = 2, 16
NW = NC * NS
B = 16384
C = 128
S = B // NW
NCH = S // C
ED = 64
PD = 32


def _sc_gather_all(idx2, tables):
    mesh = plsc.VectorSubcoreMesh(
        core_axis_name="c", subcore_axis_name="s",
        num_cores=NC, num_subcores=NS)
    dims = (2 * ED, 2 * ED, 2 * ED, PD, PD, PD, PD, PD, PD)
    tsel = (0, 1, 1, 2, 2, 3, 3, 4, 4)

    out_type = [jax.ShapeDtypeStruct((B, d), jnp.float32) for d in dims]
    scratch = (
        [pltpu.VMEM((NCH, C), jnp.int32) for _ in range(9)]
        + [pltpu.VMEM((C, d), jnp.float32) for d in dims]
        + [pltpu.SemaphoreType.DMA]
    )

    @functools.partial(
        pl.kernel, mesh=mesh, out_type=out_type, scratch_types=scratch,
        name="bpr_sc_gather",
        compiler_params=pltpu.CompilerParams(use_tc_tiling_on_sc=False),
    )
    def k(*refs):
        idx_hbm = refs[0:9]
        tab_hbm = refs[9:14]
        out_hbm = refs[14:23]
        idx_v = refs[23:32]
        row_v = refs[32:41]
        sem = refs[41]

        wid = lax.axis_index("s") * NC + lax.axis_index("c")
        row0 = wid * NCH

        for t in range(9):
            pltpu.sync_copy(idx_hbm[t].at[pl.ds(row0, NCH)], idx_v[t])

        def chunk(g):
            off = pl.multiple_of((row0 + g) * C, C)
            handles = [
                pltpu.async_copy(
                    tab_hbm[tsel[t]].at[idx_v[t].at[g]], row_v[t], sem)
                for t in range(9)
            ]
            for h in handles:
                h.wait()
            for t in range(9):
                pltpu.sync_copy(row_v[t], out_hbm[t].at[pl.ds(off, C)])

        pl.loop(0, NCH)(chunk)

    return k(*idx2, *tables)


K = 1 << 19
_BN = 2048


def _conv_body(ua_ref, ub_ref, ia_ref, ib_ref, ou_ref, oi_ref):
    eye = jnp.eye(ED, dtype=jnp.float32)
    tr = lambda x: lax.dot_general(x, eye, (((0,), (0,)), ((), ())))
    ou_ref[...] = jnp.concatenate([tr(ua_ref[...]), tr(ub_ref[...])], axis=1)
    oi_ref[...] = jnp.concatenate([tr(ia_ref[...]), tr(ib_ref[...])], axis=1)


def _tc_convert(user_table, item_table):
    grid = K // _BN
    n = user_table.shape[0]
    nlast = (n + _BN - 1) // _BN - 1
    lo = pl.BlockSpec((ED, _BN), lambda i: (0, i))
    hi = pl.BlockSpec(
        (ED, _BN), lambda i: (0, jnp.minimum(i + K // _BN, nlast)))
    out = pl.pallas_call(
        _conv_body,
        grid=(grid,),
        in_specs=[lo, hi, lo, hi],
        out_specs=[pl.BlockSpec((_BN, 2 * ED), lambda i: (i, 0))] * 2,
        out_shape=[jax.ShapeDtypeStruct((K, 2 * ED), jnp.float32)] * 2,
        compiler_params=pltpu.CompilerParams(
            fuse_transposed_lhs_in_matmul=True),
    )(user_table.T, user_table.T, item_table.T, item_table.T)
    return out


_BM = 2048


def _half(ref, par_ref):
    x = ref[...]
    p = par_ref[...]
    return jnp.where(p > 0.5, x[:, ED:2 * ED], x[:, 0:ED])


def _tc_body(u_ref, pi_ref, ni_ref, pc_ref, nc_ref, ppt_ref, npt_ref,
             pv_ref, nv_ref, pu_ref, pp_ref, pn_ref, w_ref, b_ref, o_ref):
    dot = functools.partial(lax.dot, preferred_element_type=jnp.float32)
    w1 = w_ref[0:ED, :]
    w2 = w_ref[ED:ED + PD, :]
    w3 = w_ref[ED + PD:ED + 2 * PD, :]
    w4 = w_ref[ED + 2 * PD:ED + 3 * PD, :]
    bb = b_ref[0:1, :]
    pre_p = (dot(_half(pi_ref, pp_ref), w1) + dot(pc_ref[...], w2)
             + dot(ppt_ref[...], w3) + dot(pv_ref[...], w4) + bb)
    pre_n = (dot(_half(ni_ref, pn_ref), w1) + dot(nc_ref[...], w2)
             + dot(npt_ref[...], w3) + dot(nv_ref[...], w4) + bb)
    u = _half(u_ref, pu_ref)
    s = (jnp.sum(u * jnp.maximum(pre_p, 0.0), axis=1)
         - jnp.sum(u * jnp.maximum(pre_n, 0.0), axis=1))
    o_ref[...] = s[None, None, :]


def _tc_score(rows, pars, W, b):
    grid = B // _BM
    in_specs = [
        pl.BlockSpec((_BM, r.shape[1]), lambda i: (i, 0)) for r in rows
    ] + [
        pl.BlockSpec((_BM, 1), lambda i: (i, 0)) for _ in pars
    ] + [
        pl.BlockSpec((ED + 3 * PD, ED), lambda i: (0, 0)),
        pl.BlockSpec((1, ED), lambda i: (0, 0)),
    ]
    out = pl.pallas_call(
        _tc_body,
        grid=(grid,),
        in_specs=in_specs,
        out_specs=pl.BlockSpec((1, 1, _BM), lambda i: (i, 0, 0)),
        out_shape=jax.ShapeDtypeStruct((grid, 1, _BM), jnp.float32),
    )(*rows, *pars, W, b.reshape(1, ED))
    return out.reshape(B)


def kernel(user_ids, pos_item_ids, neg_item_ids, pos_cat, neg_cat,
           pos_prop_type, pos_prop_value, neg_prop_type, neg_prop_value,
           user_table, item_table, cat_table, ptype_table, pval_table, W, b):
    big = [user_ids.astype(jnp.int32), pos_item_ids.astype(jnp.int32),
           neg_item_ids.astype(jnp.int32)]
    small = [pos_cat, neg_cat, pos_prop_type, neg_prop_type,
             pos_prop_value, neg_prop_value]
    idx2 = ([(i & (K - 1)).reshape(B // C, C) for i in big]
            + [i.astype(jnp.int32).reshape(B // C, C) for i in small])
    pars = [(i >> 19).astype(jnp.float32).reshape(B, 1) for i in big]
    user_pk, item_pk = _tc_convert(user_table, item_table)
    tables = [user_pk, item_pk, cat_table, ptype_table, pval_table]
    rows = _sc_gather_all(idx2, tables)
    return _tc_score(rows, pars, W, b)


# speedup vs baseline: 2.3045x; 1.0809x over previous
import functools

import jax
import jax.numpy as jnp
from jax import lax
from jax.experimental import pallas as pl
from jax.experimental.pallas import tpu as pltpu
from jax.experimental.pallas import tpu_sc as plsc

NC, NS = 2, 16
NW = NC * NS
B = 16384
C = 128
S = B // NW
NCH = S // C
ED = 64
PD = 32


def _sc_gather_all(idx2, tables):
    mesh = plsc.VectorSubcoreMesh(
        core_axis_name="c", subcore_axis_name="s",
        num_cores=NC, num_subcores=NS)
    dims = (2 * ED, 2 * ED, 2 * ED, PD, PD, PD, PD, PD, PD)
    tsel = (0, 1, 1, 2, 2, 3, 3, 4, 4)

    out_type = [jax.ShapeDtypeStruct((B, d), jnp.float32) for d in dims]
    scratch = (
        [pltpu.VMEM((NCH, C), jnp.int32) for _ in range(9)]
        + [pltpu.VMEM((C, d), jnp.float32) for d in dims]
        + [pltpu.SemaphoreType.DMA]
    )

    @functools.partial(
        pl.kernel, mesh=mesh, out_type=out_type, scratch_types=scratch,
        name="bpr_sc_gather",
        compiler_params=pltpu.CompilerParams(use_tc_tiling_on_sc=False),
    )
    def k(*refs):
        idx_hbm = refs[0:9]
        tab_hbm = refs[9:14]
        out_hbm = refs[14:23]
        idx_v = refs[23:32]
        row_v = refs[32:41]
        sem = refs[41]

        wid = lax.axis_index("s") * NC + lax.axis_index("c")
        row0 = wid * NCH

        for t in range(9):
            pltpu.sync_copy(idx_hbm[t].at[pl.ds(row0, NCH)], idx_v[t])

        def chunk(g):
            off = pl.multiple_of((row0 + g) * C, C)
            handles = [
                pltpu.async_copy(
                    tab_hbm[tsel[t]].at[idx_v[t].at[g]], row_v[t], sem)
                for t in range(9)
            ]
            for h in handles:
                h.wait()
            for t in range(9):
                pltpu.sync_copy(row_v[t], out_hbm[t].at[pl.ds(off, C)])

        pl.loop(0, NCH)(chunk)

    return k(*idx2, *tables)


K = 1 << 19
_BN = 2048


def _conv_body(ua_ref, ub_ref, ia_ref, ib_ref, ou_ref, oi_ref):
    eye = jnp.eye(ED, dtype=jnp.bfloat16)

    def tr(x):
        return lax.dot_general(
            x.astype(jnp.bfloat16), eye, (((0,), (0,)), ((), ())),
            preferred_element_type=jnp.float32)
    ou_ref[...] = jnp.concatenate([tr(ua_ref[...]), tr(ub_ref[...])], axis=1)
    oi_ref[...] = jnp.concatenate([tr(ia_ref[...]), tr(ib_ref[...])], axis=1)


def _tc_convert(user_table, item_table):
    grid = K // _BN
    n = user_table.shape[0]
    nlast = (n + _BN - 1) // _BN - 1
    lo = pl.BlockSpec((ED, _BN), lambda i: (0, i))
    hi = pl.BlockSpec(
        (ED, _BN), lambda i: (0, jnp.minimum(i + K // _BN, nlast)))
    out = pl.pallas_call(
        _conv_body,
        grid=(grid,),
        in_specs=[lo, hi, lo, hi],
        out_specs=[pl.BlockSpec((_BN, 2 * ED), lambda i: (i, 0))] * 2,
        out_shape=[jax.ShapeDtypeStruct((K, 2 * ED), jnp.float32)] * 2,
        compiler_params=pltpu.CompilerParams(
            fuse_transposed_lhs_in_matmul=True),
    )(user_table.T, user_table.T, item_table.T, item_table.T)
    return out


_BM = 2048


def _half(ref, par_ref):
    x = ref[...]
    p = par_ref[...]
    return jnp.where(p > 0.5, x[:, ED:2 * ED], x[:, 0:ED])


def _tc_body(u_ref, pi_ref, ni_ref, pc_ref, nc_ref, ppt_ref, npt_ref,
             pv_ref, nv_ref, pu_ref, pp_ref, pn_ref, w_ref, b_ref, o_ref):
    dot = functools.partial(lax.dot, preferred_element_type=jnp.float32)
    w1 = w_ref[0:ED, :]
    w2 = w_ref[ED:ED + PD, :]
    w3 = w_ref[ED + PD:ED + 2 * PD, :]
    w4 = w_ref[ED + 2 * PD:ED + 3 * PD, :]
    bb = b_ref[0:1, :]
    pre_p = (dot(_half(pi_ref, pp_ref), w1) + dot(pc_ref[...], w2)
             + dot(ppt_ref[...], w3) + dot(pv_ref[...], w4) + bb)
    pre_n = (dot(_half(ni_ref, pn_ref), w1) + dot(nc_ref[...], w2)
             + dot(npt_ref[...], w3) + dot(nv_ref[...], w4) + bb)
    u = _half(u_ref, pu_ref)
    s = (jnp.sum(u * jnp.maximum(pre_p, 0.0), axis=1)
         - jnp.sum(u * jnp.maximum(pre_n, 0.0), axis=1))
    o_ref[...] = s[None, None, :]


def _tc_score(rows, pars, W, b):
    grid = B // _BM
    in_specs = [
        pl.BlockSpec((_BM, r.shape[1]), lambda i: (i, 0)) for r in rows
    ] + [
        pl.BlockSpec((_BM, 1), lambda i: (i, 0)) for _ in pars
    ] + [
        pl.BlockSpec((ED + 3 * PD, ED), lambda i: (0, 0)),
        pl.BlockSpec((1, ED), lambda i: (0, 0)),
    ]
    out = pl.pallas_call(
        _tc_body,
        grid=(grid,),
        in_specs=in_specs,
        out_specs=pl.BlockSpec((1, 1, _BM), lambda i: (i, 0, 0)),
        out_shape=jax.ShapeDtypeStruct((grid, 1, _BM), jnp.float32),
    )(*rows, *pars, W, b.reshape(1, ED))
    return out.reshape(B)


def kernel(user_ids, pos_item_ids, neg_item_ids, pos_cat, neg_cat,
           pos_prop_type, pos_prop_value, neg_prop_type, neg_prop_value,
           user_table, item_table, cat_table, ptype_table, pval_table, W, b):
    big = [user_ids.astype(jnp.int32), pos_item_ids.astype(jnp.int32),
           neg_item_ids.astype(jnp.int32)]
    small = [pos_cat, neg_cat, pos_prop_type, neg_prop_type,
             pos_prop_value, neg_prop_value]
    idx2 = ([(i & (K - 1)).reshape(B // C, C) for i in big]
            + [i.astype(jnp.int32).reshape(B // C, C) for i in small])
    pars = [(i >> 19).astype(jnp.float32).reshape(B, 1) for i in big]
    user_pk, item_pk = _tc_convert(user_table, item_table)
    tables = [user_pk, item_pk, cat_table, ptype_table, pval_table]
    rows = _sc_gather_all(idx2, tables)
    return _tc_score(rows, pars, W, b)


# speedup vs baseline: 2.3756x; 1.0309x over previous
import functools

import jax
import jax.numpy as jnp
from jax import lax
from jax.experimental import pallas as pl
from jax.experimental.pallas import tpu as pltpu
from jax.experimental.pallas import tpu_sc as plsc

NC, NS = 2, 16
NW = NC * NS
B = 16384
C = 128
S = B // NW
NCH = S // C
ED = 64
HD = ED // 2
PD = 32
K4 = 1 << 18


def _sc_gather(idx2, tables, tsel, dims, dts, name):
    mesh = plsc.VectorSubcoreMesh(
        core_axis_name="c", subcore_axis_name="s",
        num_cores=NC, num_subcores=NS)
    ng = len(tsel)
    nt = len(tables)

    out_type = [jax.ShapeDtypeStruct((B, d), dt) for d, dt in zip(dims, dts)]
    scratch = (
        [pltpu.VMEM((NCH, C), jnp.int32) for _ in range(ng)]
        + [pltpu.VMEM((C, d), dt) for d, dt in zip(dims, dts)]
        + [pltpu.SemaphoreType.DMA]
    )

    @functools.partial(
        pl.kernel, mesh=mesh, out_type=out_type, scratch_types=scratch,
        name=name,
        compiler_params=pltpu.CompilerParams(use_tc_tiling_on_sc=False),
    )
    def k(*refs):
        idx_hbm = refs[0:ng]
        tab_hbm = refs[ng:ng + nt]
        out_hbm = refs[ng + nt:2 * ng + nt]
        idx_v = refs[2 * ng + nt:3 * ng + nt]
        row_v = refs[3 * ng + nt:4 * ng + nt]
        sem = refs[4 * ng + nt]

        wid = lax.axis_index("s") * NC + lax.axis_index("c")
        row0 = wid * NCH

        for t in range(ng):
            pltpu.sync_copy(idx_hbm[t].at[pl.ds(row0, NCH)], idx_v[t])

        def chunk(g):
            off = pl.multiple_of((row0 + g) * C, C)
            handles = [
                pltpu.async_copy(
                    tab_hbm[tsel[t]].at[idx_v[t].at[g]], row_v[t], sem)
                for t in range(ng)
            ]
            for h in handles:
                h.wait()
            for t in range(ng):
                pltpu.sync_copy(row_v[t], out_hbm[t].at[pl.ds(off, C)])

        pl.loop(0, NCH)(chunk)

    return k(*idx2, *tables)


_BN = 2048


def _conv_body(a_ref, b_ref, c_ref, d_ref, o_ref):
    eye = jnp.eye(ED, dtype=jnp.bfloat16)
    e_lo, e_hi = eye[:, 0:HD], eye[:, HD:ED]

    def quarter(x_ref):
        x = x_ref[...].astype(jnp.bfloat16)
        dn = (((0,), (0,)), ((), ()))
        lo = lax.dot_general(x, e_lo, dn, preferred_element_type=jnp.float32)
        hi = lax.dot_general(x, e_hi, dn, preferred_element_type=jnp.float32)
        lob = lax.shift_right_logical(
            lax.bitcast_convert_type(lo, jnp.int32), 16)
        hib = lax.bitcast_convert_type(hi, jnp.int32)
        return hib | lob

    o_ref[...] = jnp.concatenate(
        [quarter(r) for r in (a_ref, b_ref, c_ref, d_ref)], axis=1)


def _tc_convert(table):
    n = table.shape[0]
    grid = K4 // _BN
    nlast = (n + _BN - 1) // _BN - 1
    qoff = K4 // _BN

    def qspec(q):
        if q == 0:
            return pl.BlockSpec((ED, _BN), lambda i: (0, i))
        return pl.BlockSpec(
            (ED, _BN),
            lambda i, q=q: (0, jnp.minimum(i + q * qoff, nlast)))

    return pl.pallas_call(
        _conv_body,
        grid=(grid,),
        in_specs=[qspec(q) for q in range(4)],
        out_specs=pl.BlockSpec((_BN, 2 * ED), lambda i: (i, 0)),
        out_shape=jax.ShapeDtypeStruct((K4, 2 * ED), jnp.int32),
        compiler_params=pltpu.CompilerParams(
            fuse_transposed_lhs_in_matmul=True),
    )(table.T, table.T, table.T, table.T)


_BM = 2048


def _unpack(ref, q_ref):
    x = ref[...]
    q = q_ref[...]
    s = jnp.where(
        q < 1.5,
        jnp.where(q < 0.5, x[:, 0:32], x[:, 32:64]),
        jnp.where(q < 2.5, x[:, 64:96], x[:, 96:128]))
    lo = lax.bitcast_convert_type(lax.shift_left(s, 16), jnp.float32)
    hi = lax.bitcast_convert_type(
        lax.shift_left(lax.shift_right_logical(s, 16), 16), jnp.float32)
    return lo, hi


def _tc_body(u_ref, pi_ref, ni_ref, pc_ref, nc_ref, ppt_ref, npt_ref,
             pv_ref, nv_ref, qu_ref, qp_ref, qn_ref, wa_ref, wb_ref,
             ba_ref, bb_ref, o_ref):
    dot = functools.partial(lax.dot, preferred_element_type=jnp.float32)

    def pre(it_lo, it_hi, c_ref, pt_ref, v_ref, w_ref, b_ref):
        return (dot(it_lo, w_ref[0:HD, :]) + dot(it_hi, w_ref[HD:ED, :])
                + dot(c_ref[...], w_ref[ED:ED + PD, :])
                + dot(pt_ref[...], w_ref[ED + PD:ED + 2 * PD, :])
                + dot(v_ref[...], w_ref[ED + 2 * PD:ED + 3 * PD, :])
                + b_ref[0:1, :])

    pi_lo, pi_hi = _unpack(pi_ref, qp_ref)
    ni_lo, ni_hi = _unpack(ni_ref, qn_ref)
    u_lo, u_hi = _unpack(u_ref, qu_ref)

    z = jnp.float32(0.0)
    rp_a = jnp.maximum(pre(pi_lo, pi_hi, pc_ref, ppt_ref, pv_ref,
                           wa_ref, ba_ref), z)
    rp_b = jnp.maximum(pre(pi_lo, pi_hi, pc_ref, ppt_ref, pv_ref,
                           wb_ref, bb_ref), z)
    rn_a = jnp.maximum(pre(ni_lo, ni_hi, nc_ref, npt_ref, nv_ref,
                           wa_ref, ba_ref), z)
    rn_b = jnp.maximum(pre(ni_lo, ni_hi, nc_ref, npt_ref, nv_ref,
                           wb_ref, bb_ref), z)
    s = (jnp.sum(u_lo * (rp_a - rn_a), axis=1)
         + jnp.sum(u_hi * (rp_b - rn_b), axis=1))
    o_ref[...] = s[None, None, :]


def _tc_score(rows, quarters, W, b):
    grid = B // _BM
    in_specs = [
        pl.BlockSpec((_BM, r.shape[1]), lambda i: (i, 0)) for r in rows
    ] + [
        pl.BlockSpec((_BM, 1), lambda i: (i, 0)) for _ in quarters
    ] + [
        pl.BlockSpec((ED + 3 * PD, HD), lambda i: (0, 0)),
        pl.BlockSpec((ED + 3 * PD, HD), lambda i: (0, 0)),
        pl.BlockSpec((1, HD), lambda i: (0, 0)),
        pl.BlockSpec((1, HD), lambda i: (0, 0)),
    ]
    out = pl.pallas_call(
        _tc_body,
        grid=(grid,),
        in_specs=in_specs,
        out_specs=pl.BlockSpec((1, 1, _BM), lambda i: (i, 0, 0)),
        out_shape=jax.ShapeDtypeStruct((grid, 1, _BM), jnp.float32),
    )(*rows, *quarters, W[:, 0:HD], W[:, HD:ED],
      b[0:HD].reshape(1, HD), b[HD:ED].reshape(1, HD))
    return out.reshape(B)


def kernel(user_ids, pos_item_ids, neg_item_ids, pos_cat, neg_cat,
           pos_prop_type, pos_prop_value, neg_prop_type, neg_prop_value,
           user_table, item_table, cat_table, ptype_table, pval_table, W, b):
    big = [user_ids.astype(jnp.int32), pos_item_ids.astype(jnp.int32),
           neg_item_ids.astype(jnp.int32)]
    small = [pos_cat, neg_cat, pos_prop_type, neg_prop_type,
             pos_prop_value, neg_prop_value]
    bidx = [(i & (K4 - 1)).reshape(B // C, C) for i in big]
    sidx = [i.astype(jnp.int32).reshape(B // C, C) for i in small]
    quarters = [(i >> 18).astype(jnp.float32).reshape(B, 1) for i in big]

    item_pk = _tc_convert(item_table)
    user_pk = _tc_convert(user_table)

    i32t, f32t = jnp.int32, jnp.float32
    r_items = _sc_gather(
        bidx[1:3] + sidx,
        [item_pk, cat_table, ptype_table, pval_table],
        (0, 0, 1, 1, 2, 2, 3, 3),
        (2 * ED, 2 * ED, PD, PD, PD, PD, PD, PD),
        (i32t, i32t, f32t, f32t, f32t, f32t, f32t, f32t),
        "bpr_sc_gather_items")
    r_user = _sc_gather(
        bidx[0:1], [user_pk], (0,), (2 * ED,), (i32t,), "bpr_sc_gather_user")

    rows = r_user + r_items
    return _tc_score(rows, quarters, W, b)


# speedup vs baseline: 2.8725x; 1.2092x over previous
import functools

import jax
import jax.numpy as jnp
from jax import lax
from jax.experimental import pallas as pl
from jax.experimental.pallas import tpu as pltpu
from jax.experimental.pallas import tpu_sc as plsc

NC, NS = 2, 16
NW = NC * NS
B = 16384
C = 128
S = B // NW
NCH = S // C
ED = 64
HD = ED // 2
PD = 32
K4 = 1 << 18


def _sc_gather(idx2, tables, tsel, dims, dts, name):
    mesh = plsc.VectorSubcoreMesh(
        core_axis_name="c", subcore_axis_name="s",
        num_cores=NC, num_subcores=NS)
    ng = len(tsel)
    nt = len(tables)

    out_type = [jax.ShapeDtypeStruct((B, d), dt) for d, dt in zip(dims, dts)]
    scratch = (
        [pltpu.VMEM((NCH, C), jnp.int32) for _ in range(ng)]
        + [pltpu.VMEM((C, d), dt) for d, dt in zip(dims, dts)]
        + [pltpu.SemaphoreType.DMA]
    )

    @functools.partial(
        pl.kernel, mesh=mesh, out_type=out_type, scratch_types=scratch,
        name=name,
        compiler_params=pltpu.CompilerParams(use_tc_tiling_on_sc=False),
    )
    def k(*refs):
        idx_hbm = refs[0:ng]
        tab_hbm = refs[ng:ng + nt]
        out_hbm = refs[ng + nt:2 * ng + nt]
        idx_v = refs[2 * ng + nt:3 * ng + nt]
        row_v = refs[3 * ng + nt:4 * ng + nt]
        sem = refs[4 * ng + nt]

        wid = lax.axis_index("s") * NC + lax.axis_index("c")
        row0 = wid * NCH

        for t in range(ng):
            pltpu.sync_copy(idx_hbm[t].at[pl.ds(row0, NCH)], idx_v[t])

        def chunk(g):
            off = pl.multiple_of((row0 + g) * C, C)
            handles = [
                pltpu.async_copy(
                    tab_hbm[tsel[t]].at[idx_v[t].at[g]], row_v[t], sem)
                for t in range(ng)
            ]
            for h in handles:
                h.wait()
            for t in range(ng):
                pltpu.sync_copy(row_v[t], out_hbm[t].at[pl.ds(off, C)])

        pl.loop(0, NCH)(chunk)

    return k(*idx2, *tables)


_BN = 4096


def _conv_body(a_ref, b_ref, c_ref, d_ref, o_ref):
    eye = jnp.eye(ED, dtype=jnp.bfloat16)
    z = jnp.zeros((ED, HD), dtype=jnp.bfloat16)
    r_lo = jnp.concatenate(
        [jnp.concatenate(
            [eye[:, 0:HD] if c == q else z for c in range(4)], axis=1)
         for q in range(4)], axis=0)
    r_hi = jnp.concatenate(
        [jnp.concatenate(
            [eye[:, HD:ED] if c == q else z for c in range(4)], axis=1)
         for q in range(4)], axis=0)
    d = d_ref[...]
    d = jnp.where(jnp.abs(d) < jnp.float32(1e30), d, jnp.float32(0.0))
    x = jnp.concatenate(
        [a_ref[...], b_ref[...], c_ref[...], d],
        axis=0).astype(jnp.bfloat16)
    dn = (((0,), (0,)), ((), ()))
    lo = lax.dot_general(x, r_lo, dn, preferred_element_type=jnp.float32)
    hi = lax.dot_general(x, r_hi, dn, preferred_element_type=jnp.float32)
    lob = lax.shift_right_logical(lax.bitcast_convert_type(lo, jnp.int32), 16)
    hib = lax.bitcast_convert_type(hi, jnp.int32)
    o_ref[...] = hib | lob


def _tc_convert(table):
    n = table.shape[0]
    grid = K4 // _BN
    nlast = (n + _BN - 1) // _BN - 1
    qoff = K4 // _BN

    def qspec(q):
        if q == 0:
            return pl.BlockSpec((ED, _BN), lambda i: (0, i))
        return pl.BlockSpec(
            (ED, _BN),
            lambda i, q=q: (0, jnp.minimum(i + q * qoff, nlast)))

    return pl.pallas_call(
        _conv_body,
        grid=(grid,),
        in_specs=[qspec(q) for q in range(4)],
        out_specs=pl.BlockSpec((_BN, 2 * ED), lambda i: (i, 0)),
        out_shape=jax.ShapeDtypeStruct((K4, 2 * ED), jnp.int32),
        compiler_params=pltpu.CompilerParams(
            fuse_transposed_lhs_in_matmul=True),
    )(table.T, table.T, table.T, table.T)


_BM = 2048


def _unpack(ref, q_ref):
    x = ref[...]
    q = q_ref[...]
    s = jnp.where(
        q < 1.5,
        jnp.where(q < 0.5, x[:, 0:32], x[:, 32:64]),
        jnp.where(q < 2.5, x[:, 64:96], x[:, 96:128]))
    lo = lax.bitcast_convert_type(lax.shift_left(s, 16), jnp.float32)
    hi = lax.bitcast_convert_type(
        lax.shift_left(lax.shift_right_logical(s, 16), 16), jnp.float32)
    return lo, hi


def _tc_body(u_ref, pi_ref, ni_ref, pc_ref, nc_ref, ppt_ref, npt_ref,
             pv_ref, nv_ref, qu_ref, qp_ref, qn_ref, wa_ref, wb_ref,
             ba_ref, bb_ref, o_ref):
    dot = functools.partial(lax.dot, preferred_element_type=jnp.float32)

    def pre(it_lo, it_hi, c_ref, pt_ref, v_ref, w_ref, b_ref):
        return (dot(it_lo, w_ref[0:HD, :]) + dot(it_hi, w_ref[HD:ED, :])
                + dot(c_ref[...], w_ref[ED:ED + PD, :])
                + dot(pt_ref[...], w_ref[ED + PD:ED + 2 * PD, :])
                + dot(v_ref[...], w_ref[ED + 2 * PD:ED + 3 * PD, :])
                + b_ref[0:1, :])

    pi_lo, pi_hi = _unpack(pi_ref, qp_ref)
    ni_lo, ni_hi = _unpack(ni_ref, qn_ref)
    u_lo, u_hi = _unpack(u_ref, qu_ref)

    z = jnp.float32(0.0)
    rp_a = jnp.maximum(pre(pi_lo, pi_hi, pc_ref, ppt_ref, pv_ref,
                           wa_ref, ba_ref), z)
    rp_b = jnp.maximum(pre(pi_lo, pi_hi, pc_ref, ppt_ref, pv_ref,
                           wb_ref, bb_ref), z)
    rn_a = jnp.maximum(pre(ni_lo, ni_hi, nc_ref, npt_ref, nv_ref,
                           wa_ref, ba_ref), z)
    rn_b = jnp.maximum(pre(ni_lo, ni_hi, nc_ref, npt_ref, nv_ref,
                           wb_ref, bb_ref), z)
    s = (jnp.sum(u_lo * (rp_a - rn_a), axis=1)
         + jnp.sum(u_hi * (rp_b - rn_b), axis=1))
    o_ref[...] = s[None, None, :]


def _tc_score(rows, quarters, W, b):
    grid = B // _BM
    in_specs = [
        pl.BlockSpec((_BM, r.shape[1]), lambda i: (i, 0)) for r in rows
    ] + [
        pl.BlockSpec((_BM, 1), lambda i: (i, 0)) for _ in quarters
    ] + [
        pl.BlockSpec((ED + 3 * PD, HD), lambda i: (0, 0)),
        pl.BlockSpec((ED + 3 * PD, HD), lambda i: (0, 0)),
        pl.BlockSpec((1, HD), lambda i: (0, 0)),
        pl.BlockSpec((1, HD), lambda i: (0, 0)),
    ]
    out = pl.pallas_call(
        _tc_body,
        grid=(grid,),
        in_specs=in_specs,
        out_specs=pl.BlockSpec((1, 1, _BM), lambda i: (i, 0, 0)),
        out_shape=jax.ShapeDtypeStruct((grid, 1, _BM), jnp.float32),
    )(*rows, *quarters, W[:, 0:HD], W[:, HD:ED],
      b[0:HD].reshape(1, HD), b[HD:ED].reshape(1, HD))
    return out.reshape(B)


def kernel(user_ids, pos_item_ids, neg_item_ids, pos_cat, neg_cat,
           pos_prop_type, pos_prop_value, neg_prop_type, neg_prop_value,
           user_table, item_table, cat_table, ptype_table, pval_table, W, b):
    big = [user_ids.astype(jnp.int32), pos_item_ids.astype(jnp.int32),
           neg_item_ids.astype(jnp.int32)]
    small = [pos_cat, neg_cat, pos_prop_type, neg_prop_type,
             pos_prop_value, neg_prop_value]
    bidx = [(i & (K4 - 1)).reshape(B // C, C) for i in big]
    sidx = [i.astype(jnp.int32).reshape(B // C, C) for i in small]
    quarters = [(i >> 18).astype(jnp.float32).reshape(B, 1) for i in big]

    item_pk = _tc_convert(item_table)
    user_pk = _tc_convert(user_table)

    i32t, f32t = jnp.int32, jnp.float32
    r_small = _sc_gather(
        sidx, [cat_table, ptype_table, pval_table],
        (0, 0, 1, 1, 2, 2), (PD,) * 6, (f32t,) * 6, "bpr_sc_gather_small")
    r_items = _sc_gather(
        bidx[1:3], [item_pk], (0, 0), (2 * ED, 2 * ED), (i32t, i32t),
        "bpr_sc_gather_items")
    r_user = _sc_gather(
        bidx[0:1], [user_pk], (0,), (2 * ED,), (i32t,), "bpr_sc_gather_user")

    rows = r_user + r_items + r_small
    return _tc_score(rows, quarters, W, b)
